# baseline (speedup 1.0000x reference)
import functools

import jax
import jax.numpy as jnp
from jax import lax
from jax.experimental import pallas as pl
from jax.experimental.pallas import tpu as pltpu
from jax.experimental.pallas import tpu_sc as plsc

HID = 128
N_NODES = 50000
N_EDGES = 800000
NGRAPHS = 1024

NC = 2
NS = 16
NW = NC * NS

NPADR = 50176
RBLK = 1568
NCHUNK = 4
CH = 12544
CSL = CH // NS
EPAD = 819200
EQ = EPAD // NW
EB = 2560
NBUF = 4
PNB = 7
INVALID_DST = 1 << 30

@functools.lru_cache(maxsize=None)
def _sc_mesh():
    return plsc.VectorSubcoreMesh(
        core_axis_name="c", subcore_axis_name="s",
        num_cores=NC, num_subcores=NS)


def _agg(src, dst, x):
    return _make_agg()(src, dst, x)


@functools.lru_cache(maxsize=None)
def _make_agg():
    return functools.partial(
        pl.kernel,
        out_type=jax.ShapeDtypeStruct((NC, NPADR, HID), jnp.float32),
        mesh=_sc_mesh(),
        scratch_types=[
            pltpu.VMEM((EB,), jnp.int32),
            pltpu.VMEM((EB,), jnp.int32),
            pltpu.VMEM((NBUF, 16, HID), jnp.float32),
            pltpu.VMEM((16, HID), jnp.float32),
            pltpu.VMEM_SHARED((CH + 1, HID), jnp.float32),
            pltpu.SemaphoreType.DMA((NBUF,)),
            pltpu.SemaphoreType.DMA((NBUF,)),
        ],
    )(_agg_body)


def _agg_body(src_hbm, dst_hbm, x_hbm, out_hbm,
              srcbuf, dstbuf, rows, zbuf, acc, gsem, ssem):
    c = lax.axis_index("c")
    s = lax.axis_index("s")
    wid = s * NC + c
    ebase = wid * EQ

    zv = jnp.zeros((16,), jnp.float32)

    def _zrow(r, carry):
        for k in range(HID // 16):
            zbuf[r, pl.ds(k * 16, 16)] = zv
        return carry
    lax.fori_loop(0, 16, _zrow, 0)

    def _chunk(ci, carry):
        base = ci * CH
        roff = s * CSL

        def _zacc(r, carry2):
            pltpu.sync_copy(zbuf, acc.at[pl.ds(roff + r * 16, 16), :])
            return carry2
        lax.fori_loop(0, CSL // 16, _zacc, 0)
        plsc.subcore_barrier()

        def _blk(bl, carry2):
            pltpu.sync_copy(src_hbm.at[pl.ds(ebase + bl * EB, EB)], srcbuf)
            pltpu.sync_copy(dst_hbm.at[pl.ds(ebase + bl * EB, EB)], dstbuf)

            def _ring(t, carry3):
                gds = []
                for b in range(NBUF):
                    off = (t * NBUF + b) * 16
                    srcv = srcbuf[pl.ds(off, 16)]
                    gds.append(pltpu.async_copy(
                        x_hbm.at[srcv], rows.at[b], gsem.at[b]))
                sds = []
                for b in range(NBUF):
                    gds[b].wait()
                    off = (t * NBUF + b) * 16
                    dstv = dstbuf[pl.ds(off, 16)]
                    m = (dstv >= base) & (dstv < base + CH)
                    idxd = jnp.where(m, dstv - base, CH)
                    sds.append(pltpu.async_copy(
                        rows.at[b], acc.at[idxd], ssem.at[b], add=True))
                for d in sds:
                    d.wait()
                return carry3
            lax.fori_loop(0, EB // (16 * NBUF), _ring, 0)
            return carry2
        lax.fori_loop(0, EQ // EB, _blk, 0)
        plsc.subcore_barrier()
        pltpu.sync_copy(acc.at[pl.ds(roff, CSL), :],
                        out_hbm.at[c, pl.ds(base + roff, CSL), :])
        plsc.subcore_barrier()
        return carry
    lax.fori_loop(0, NCHUNK, _chunk, 0)


def _pool(x, batch, ones):
    return _make_pool()(x, batch, ones)


@functools.lru_cache(maxsize=None)
def _make_pool():
    return functools.partial(
        pl.kernel,
        out_type=(jax.ShapeDtypeStruct((NC, NGRAPHS, HID), jnp.float32),
                  jax.ShapeDtypeStruct((NC, NGRAPHS, HID), jnp.float32)),
        mesh=_sc_mesh(),
        scratch_types=[
            pltpu.VMEM((RBLK,), jnp.int32),
            pltpu.VMEM((PNB, 16, HID), jnp.float32),
            pltpu.VMEM((16, HID), jnp.float32),
            pltpu.VMEM((64, HID), jnp.float32),
            pltpu.VMEM_SHARED((NGRAPHS + 1, HID), jnp.float32),
            pltpu.VMEM_SHARED((NGRAPHS + 1, HID), jnp.float32),
            pltpu.SemaphoreType.DMA((PNB,)),
            pltpu.SemaphoreType.DMA((PNB,)),
            pltpu.SemaphoreType.DMA((PNB,)),
        ],
    )(_pool_body)


def _pool_body(x_hbm, batch_hbm, ones_hbm, sums_hbm, cnts_hbm,
               bbuf, rows, ones, zs, accs, accc, gsem, ssem, csem):
    c = lax.axis_index("c")
    s = lax.axis_index("s")
    wid = s * NC + c
    nbase = wid * RBLK
    pltpu.sync_copy(batch_hbm.at[pl.ds(nbase, RBLK)], bbuf)
    pltpu.sync_copy(ones_hbm, ones)

    zv = jnp.zeros((16,), jnp.float32)

    def _zrow(r, carry):
        for k in range(HID // 16):
            zs[r, pl.ds(k * 16, 16)] = zv
        return carry
    lax.fori_loop(0, 64, _zrow, 0)

    goff = s * (NGRAPHS // NS)
    pltpu.sync_copy(zs, accs.at[pl.ds(goff, 64), :])
    pltpu.sync_copy(zs, accc.at[pl.ds(goff, 64), :])
    plsc.subcore_barrier()

    def _ring(t, carry):
        gds = []
        for b in range(PNB):
            noff = nbase + (t * PNB + b) * 16
            gds.append(pltpu.async_copy(
                x_hbm.at[pl.ds(noff, 16), :], rows.at[b], gsem.at[b]))
        sds = []
        for b in range(PNB):
            gds[b].wait()
            idxv = bbuf[pl.ds((t * PNB + b) * 16, 16)]
            sds.append(pltpu.async_copy(
                rows.at[b], accs.at[idxv], ssem.at[b], add=True))
            sds.append(pltpu.async_copy(
                ones, accc.at[idxv], csem.at[b], add=True))
        for d in sds:
            d.wait()
        return carry
    lax.fori_loop(0, (RBLK // 16) // PNB, _ring, 0)
    plsc.subcore_barrier()
    pltpu.sync_copy(accs.at[pl.ds(goff, 64), :],
                    sums_hbm.at[c, pl.ds(goff, 64), :])
    pltpu.sync_copy(accc.at[pl.ds(goff, 64), :],
                    cnts_hbm.at[c, pl.ds(goff, 64), :])


def _embed_body(x_ref, w_ref, b_ref, o_ref):
    o_ref[...] = jnp.dot(x_ref[...], w_ref[...],
                         preferred_element_type=jnp.float32) + b_ref[...]


def _embed(x, w, b):
    nfeat = x.shape[1]
    return pl.pallas_call(
        _embed_body,
        grid=(NPADR // RBLK,),
        in_specs=[pl.BlockSpec((RBLK, nfeat), lambda i: (i, 0)),
                  pl.BlockSpec((nfeat, HID), lambda i: (0, 0)),
                  pl.BlockSpec((1, HID), lambda i: (0, 0))],
        out_specs=pl.BlockSpec((RBLK, HID), lambda i: (i, 0)),
        out_shape=jax.ShapeDtypeStruct((NPADR, HID), jnp.float32),
    )(x, w, b.reshape(1, HID))


def _mlp_body(x_ref, p_ref, eps_ref, w1_ref, b1_ref, w2_ref, b2_ref, o_ref):
    h = x_ref[...] * (eps_ref[...] + 1.0) + p_ref[0] + p_ref[1]
    a = jnp.dot(h, w1_ref[...], preferred_element_type=jnp.float32)
    a = jnp.maximum(a + b1_ref[...], 0.0)
    o = jnp.dot(a, w2_ref[...], preferred_element_type=jnp.float32)
    o_ref[...] = jnp.maximum(o + b2_ref[...], 0.0)


def _mlp(x, part, eps, w1, b1, w2, b2):
    return pl.pallas_call(
        _mlp_body,
        grid=(NPADR // RBLK,),
        in_specs=[pl.BlockSpec((RBLK, HID), lambda i: (i, 0)),
                  pl.BlockSpec((NC, RBLK, HID), lambda i: (0, i, 0)),
                  pl.BlockSpec((1, 1), lambda i: (0, 0)),
                  pl.BlockSpec((HID, 2 * HID), lambda i: (0, 0)),
                  pl.BlockSpec((1, 2 * HID), lambda i: (0, 0)),
                  pl.BlockSpec((2 * HID, HID), lambda i: (0, 0)),
                  pl.BlockSpec((1, HID), lambda i: (0, 0))],
        out_specs=pl.BlockSpec((RBLK, HID), lambda i: (i, 0)),
        out_shape=jax.ShapeDtypeStruct((NPADR, HID), jnp.float32),
    )(x, part, eps.reshape(1, 1), w1, b1.reshape(1, 2 * HID),
      w2, b2.reshape(1, HID))


def _ln(x, g, b):
    m = jnp.mean(x, axis=-1, keepdims=True)
    xm = x - m
    v = jnp.mean(xm * xm, axis=-1, keepdims=True)
    return xm / jnp.sqrt(v + 1e-5) * g + b


def _head_body(sa, ca, sb, cb, ss_, cs, temp, wia, wib, ci, gi, bi,
               wt1, ct1, wt2, ct2, wf1a, wf1b, wf1c, cf1, gf1, bf1,
               wf2, cf2, gf2, bf2, wf3t, cf3, o_ref):
    def emb(s_ref, c_ref):
        ssum = s_ref[0] + s_ref[1]
        cnt = c_ref[0][:, 0:1] + c_ref[1][:, 0:1]
        return ssum / jnp.maximum(cnt, 1.0)

    e_aq = emb(sa, ca)
    e_bs = emb(sb, cb)
    e_sol = emb(ss_, cs)
    h = (jnp.dot(e_bs, wia[...], preferred_element_type=jnp.float32)
         + jnp.dot(e_sol, wib[...], preferred_element_type=jnp.float32)
         + ci[...])
    inter = jnp.maximum(_ln(h, gi[...], bi[...]), 0.0)
    t = jnp.maximum(temp[...] * wt1[...] + ct1[...], 0.0)
    temp_emb = jnp.dot(t, wt2[...], preferred_element_type=jnp.float32) + ct2[...]
    f = (jnp.dot(e_aq, wf1a[...], preferred_element_type=jnp.float32)
         + jnp.dot(inter, wf1b[...], preferred_element_type=jnp.float32)
         + jnp.dot(temp_emb, wf1c[...], preferred_element_type=jnp.float32)
         + cf1[...])
    f = jnp.maximum(_ln(f, gf1[...], bf1[...]), 0.0)
    g2 = jnp.dot(f, wf2[...], preferred_element_type=jnp.float32) + cf2[...]
    g2 = jnp.maximum(_ln(g2, gf2[...], bf2[...]), 0.0)
    o_ref[...] = (jnp.sum(g2 * wf3t[...], axis=-1, keepdims=True) + cf3[...])


def _head(args):
    return pl.pallas_call(
        _head_body,
        out_shape=jax.ShapeDtypeStruct((NGRAPHS, 1), jnp.float32),
    )(*args)


_BN_INV = float(1.0 / (1.0 + 1e-5) ** 0.5)


def _fold_layer(lp):
    s1 = lp["g1"] * _BN_INV
    s2 = lp["g2"] * _BN_INV
    return (lp["eps"].astype(jnp.float32),
            lp["W1"] * s1[None, :], lp["b1"] * s1 + lp["c1"],
            lp["W2"] * s2[None, :], lp["b2"] * s2 + lp["c2"])


def _encoder(enc, x_pad, src, dst, batch_pad):
    x = _embed(x_pad, enc["We"], enc["be"])
    for lp in enc["layers"]:
        part = _agg(src, dst, x)
        eps, w1, b1, w2, b2 = _fold_layer(lp)
        x = _mlp(x, part, eps, w1, b1, w2, b2)
    return _pool(x, batch_pad, jnp.ones((16, HID), jnp.float32))


def kernel(drug_x, drug_edge_index, drug_batch,
           solvent_x, solvent_edge_index, solvent_batch,
           temperature, params):
    f32 = jnp.float32
    i32 = jnp.int32

    def pad_rows(x):
        return jnp.concatenate(
            [x.astype(f32), jnp.zeros((NPADR - x.shape[0], x.shape[1]), f32)])

    def pad_edges(ei):
        src = jnp.concatenate(
            [ei[0].astype(i32), jnp.zeros((EPAD - ei.shape[1],), i32)])
        dst = jnp.concatenate(
            [ei[1].astype(i32), jnp.full((EPAD - ei.shape[1],), INVALID_DST)])
        return src, dst

    def pad_batch(b):
        return jnp.concatenate(
            [b.astype(i32), jnp.full((NPADR - b.shape[0],), i32(NGRAPHS))])

    dx = pad_rows(drug_x)
    sx = pad_rows(solvent_x)
    dsrc, ddst = pad_edges(drug_edge_index)
    ssrc, sdst = pad_edges(solvent_edge_index)
    dbatch = pad_batch(drug_batch)
    sbatch = pad_batch(solvent_batch)

    sa, ca = _encoder(params["enc_aq"], dx, dsrc, ddst, dbatch)
    sb, cb = _encoder(params["enc_bs"], dx, dsrc, ddst, dbatch)
    ss_, cs = _encoder(params["enc_sol"], sx, ssrc, sdst, sbatch)

    p = params
    wi = p["Wi"]
    wf1 = p["Wf1"]
    head_args = (
        sa, ca, sb, cb, ss_, cs, temperature.astype(f32),
        wi[:HID], wi[HID:], p["ci"].reshape(1, HID),
        p["gi"].reshape(1, HID), p["bi"].reshape(1, HID),
        p["Wt1"].reshape(1, 32), p["ct1"].reshape(1, 32),
        p["Wt2"], p["ct2"].reshape(1, 32),
        wf1[:HID], wf1[HID:2 * HID], wf1[2 * HID:],
        p["cf1"].reshape(1, HID),
        p["gf1"].reshape(1, HID), p["bf1"].reshape(1, HID),
        p["Wf2"], p["cf2"].reshape(1, HID // 2),
        p["gf2"].reshape(1, HID // 2), p["bf2"].reshape(1, HID // 2),
        p["Wf3"].reshape(1, HID // 2), p["cf3"].reshape(1, 1),
    )
    return _head(head_args)


# speedup vs baseline: 1.0213x; 1.0213x over previous
import functools

import jax
import jax.numpy as jnp
from jax import lax
from jax.experimental import pallas as pl
from jax.experimental.pallas import tpu as pltpu
from jax.experimental.pallas import tpu_sc as plsc

HID = 128
N_NODES = 50000
N_EDGES = 800000
NGRAPHS = 1024

NC = 2
NS = 16
NW = NC * NS

NPADR = 50176
RBLK = 1568
NCHUNK = 4
CH = 12544
CSL = CH // NS
EPAD = 819200
EQ = EPAD // NW
EB = 2560
NBUF = 2
PNB = 7
INVALID_DST = 1 << 30

@functools.lru_cache(maxsize=None)
def _sc_mesh():
    return plsc.VectorSubcoreMesh(
        core_axis_name="c", subcore_axis_name="s",
        num_cores=NC, num_subcores=NS)


def _agg(src, dst, x):
    return _make_agg()(src, dst, x)


GR = 64
NTB = EB // GR
NDUM = 16


@functools.lru_cache(maxsize=None)
def _make_agg():
    return functools.partial(
        pl.kernel,
        out_type=jax.ShapeDtypeStruct((NC, NPADR, HID), jnp.float32),
        mesh=_sc_mesh(),
        scratch_types=[
            pltpu.VMEM((EB,), jnp.int32),
            pltpu.VMEM((EB,), jnp.int32),
            pltpu.VMEM((NTB, GR), jnp.int32),
            pltpu.VMEM((NBUF, GR, HID), jnp.float32),
            pltpu.VMEM((16, HID), jnp.float32),
            pltpu.VMEM_SHARED((CH + NDUM, HID), jnp.float32),
            pltpu.SemaphoreType.DMA((NBUF,)),
            pltpu.SemaphoreType.DMA((NBUF,)),
        ],
    )(_agg_body)


def _agg_body(src_hbm, dst_hbm, x_hbm, out_hbm,
              ssbuf, sdbuf, idx2, rows, zbuf, acc, gsem, ssem):
    c = lax.axis_index("c")
    s = lax.axis_index("s")
    wid = s * NC + c
    ebase = wid * EQ
    lane = lax.iota(jnp.int32, 16)

    zv = jnp.zeros((16,), jnp.float32)

    def _zrow(r, carry):
        for k in range(HID // 16):
            zbuf[r, pl.ds(k * 16, 16)] = zv
        return carry
    lax.fori_loop(0, 16, _zrow, 0)

    def _chunk(ci, carry):
        base = ci * CH
        roff = s * CSL

        def _zacc(r, carry2):
            pltpu.sync_copy(zbuf, acc.at[pl.ds(roff + r * 16, 16), :])
            return carry2
        lax.fori_loop(0, CSL // 16, _zacc, 0)
        plsc.subcore_barrier()

        def _blk(bl, carry2):
            pltpu.sync_copy(src_hbm.at[pl.ds(ebase + bl * EB, EB)], ssbuf)
            pltpu.sync_copy(dst_hbm.at[pl.ds(ebase + bl * EB, EB)], sdbuf)

            def _repack(v, carry3):
                sdv = sdbuf[pl.ds(v * 16, 16)]
                m = (sdv >= base) & (sdv < base + CH)
                idx2[v // (GR // 16), pl.ds((v % (GR // 16)) * 16, 16)] = (
                    jnp.where(m, sdv - base, CH + lane))
                return carry3
            lax.fori_loop(0, EB // 16, _repack, 0)

            def _pair(t, carry3):
                gds = []
                for b in range(NBUF):
                    j = t * NBUF + b
                    gds.append(pltpu.async_copy(
                        x_hbm.at[ssbuf.at[pl.ds(j * GR, GR)]],
                        rows.at[b], gsem.at[b]))
                sds = []
                for b in range(NBUF):
                    gds[b].wait()
                    j = t * NBUF + b
                    sds.append(pltpu.async_copy(
                        rows.at[b], acc.at[idx2.at[j]], ssem.at[b], add=True))
                for d in sds:
                    d.wait()
                return carry3
            lax.fori_loop(0, NTB // NBUF, _pair, 0)
            return carry2
        lax.fori_loop(0, EQ // EB, _blk, 0)
        plsc.subcore_barrier()
        pltpu.sync_copy(acc.at[pl.ds(roff, CSL), :],
                        out_hbm.at[c, pl.ds(base + roff, CSL), :])
        plsc.subcore_barrier()
        return carry
    lax.fori_loop(0, NCHUNK, _chunk, 0)


def _pool(x, batch, ones):
    return _make_pool()(x, batch, ones)


@functools.lru_cache(maxsize=None)
def _make_pool():
    return functools.partial(
        pl.kernel,
        out_type=(jax.ShapeDtypeStruct((NC, NGRAPHS, HID), jnp.float32),
                  jax.ShapeDtypeStruct((NC, NGRAPHS, HID), jnp.float32)),
        mesh=_sc_mesh(),
        scratch_types=[
            pltpu.VMEM((RBLK,), jnp.int32),
            pltpu.VMEM((PNB, 16, HID), jnp.float32),
            pltpu.VMEM((16, HID), jnp.float32),
            pltpu.VMEM((64, HID), jnp.float32),
            pltpu.VMEM_SHARED((NGRAPHS + 1, HID), jnp.float32),
            pltpu.VMEM_SHARED((NGRAPHS + 1, HID), jnp.float32),
            pltpu.SemaphoreType.DMA((PNB,)),
            pltpu.SemaphoreType.DMA((PNB,)),
            pltpu.SemaphoreType.DMA((PNB,)),
        ],
    )(_pool_body)


def _pool_body(x_hbm, batch_hbm, ones_hbm, sums_hbm, cnts_hbm,
               bbuf, rows, ones, zs, accs, accc, gsem, ssem, csem):
    c = lax.axis_index("c")
    s = lax.axis_index("s")
    wid = s * NC + c
    nbase = wid * RBLK
    pltpu.sync_copy(batch_hbm.at[pl.ds(nbase, RBLK)], bbuf)
    pltpu.sync_copy(ones_hbm, ones)

    zv = jnp.zeros((16,), jnp.float32)

    def _zrow(r, carry):
        for k in range(HID // 16):
            zs[r, pl.ds(k * 16, 16)] = zv
        return carry
    lax.fori_loop(0, 64, _zrow, 0)

    goff = s * (NGRAPHS // NS)
    pltpu.sync_copy(zs, accs.at[pl.ds(goff, 64), :])
    pltpu.sync_copy(zs, accc.at[pl.ds(goff, 64), :])
    plsc.subcore_barrier()

    def _ring(t, carry):
        gds = []
        for b in range(PNB):
            noff = nbase + (t * PNB + b) * 16
            gds.append(pltpu.async_copy(
                x_hbm.at[pl.ds(noff, 16), :], rows.at[b], gsem.at[b]))
        sds = []
        for b in range(PNB):
            gds[b].wait()
            idxv = bbuf[pl.ds((t * PNB + b) * 16, 16)]
            sds.append(pltpu.async_copy(
                rows.at[b], accs.at[idxv], ssem.at[b], add=True))
            sds.append(pltpu.async_copy(
                ones, accc.at[idxv], csem.at[b], add=True))
        for d in sds:
            d.wait()
        return carry
    lax.fori_loop(0, (RBLK // 16) // PNB, _ring, 0)
    plsc.subcore_barrier()
    pltpu.sync_copy(accs.at[pl.ds(goff, 64), :],
                    sums_hbm.at[c, pl.ds(goff, 64), :])
    pltpu.sync_copy(accc.at[pl.ds(goff, 64), :],
                    cnts_hbm.at[c, pl.ds(goff, 64), :])


def _embed_body(x_ref, w_ref, b_ref, o_ref):
    o_ref[...] = jnp.dot(x_ref[...], w_ref[...],
                         preferred_element_type=jnp.float32) + b_ref[...]


def _embed(x, w, b):
    nfeat = x.shape[1]
    return pl.pallas_call(
        _embed_body,
        grid=(NPADR // RBLK,),
        in_specs=[pl.BlockSpec((RBLK, nfeat), lambda i: (i, 0)),
                  pl.BlockSpec((nfeat, HID), lambda i: (0, 0)),
                  pl.BlockSpec((1, HID), lambda i: (0, 0))],
        out_specs=pl.BlockSpec((RBLK, HID), lambda i: (i, 0)),
        out_shape=jax.ShapeDtypeStruct((NPADR, HID), jnp.float32),
    )(x, w, b.reshape(1, HID))


def _mlp_body(x_ref, p_ref, eps_ref, w1_ref, b1_ref, w2_ref, b2_ref, o_ref):
    h = x_ref[...] * (eps_ref[...] + 1.0) + p_ref[0] + p_ref[1]
    a = jnp.dot(h, w1_ref[...], preferred_element_type=jnp.float32)
    a = jnp.maximum(a + b1_ref[...], 0.0)
    o = jnp.dot(a, w2_ref[...], preferred_element_type=jnp.float32)
    o_ref[...] = jnp.maximum(o + b2_ref[...], 0.0)


def _mlp(x, part, eps, w1, b1, w2, b2):
    return pl.pallas_call(
        _mlp_body,
        grid=(NPADR // RBLK,),
        in_specs=[pl.BlockSpec((RBLK, HID), lambda i: (i, 0)),
                  pl.BlockSpec((NC, RBLK, HID), lambda i: (0, i, 0)),
                  pl.BlockSpec((1, 1), lambda i: (0, 0)),
                  pl.BlockSpec((HID, 2 * HID), lambda i: (0, 0)),
                  pl.BlockSpec((1, 2 * HID), lambda i: (0, 0)),
                  pl.BlockSpec((2 * HID, HID), lambda i: (0, 0)),
                  pl.BlockSpec((1, HID), lambda i: (0, 0))],
        out_specs=pl.BlockSpec((RBLK, HID), lambda i: (i, 0)),
        out_shape=jax.ShapeDtypeStruct((NPADR, HID), jnp.float32),
    )(x, part, eps.reshape(1, 1), w1, b1.reshape(1, 2 * HID),
      w2, b2.reshape(1, HID))


def _ln(x, g, b):
    m = jnp.mean(x, axis=-1, keepdims=True)
    xm = x - m
    v = jnp.mean(xm * xm, axis=-1, keepdims=True)
    return xm / jnp.sqrt(v + 1e-5) * g + b


def _head_body(sa, ca, sb, cb, ss_, cs, temp, wia, wib, ci, gi, bi,
               wt1, ct1, wt2, ct2, wf1a, wf1b, wf1c, cf1, gf1, bf1,
               wf2, cf2, gf2, bf2, wf3t, cf3, o_ref):
    def emb(s_ref, c_ref):
        ssum = s_ref[0] + s_ref[1]
        cnt = c_ref[0][:, 0:1] + c_ref[1][:, 0:1]
        return ssum / jnp.maximum(cnt, 1.0)

    e_aq = emb(sa, ca)
    e_bs = emb(sb, cb)
    e_sol = emb(ss_, cs)
    h = (jnp.dot(e_bs, wia[...], preferred_element_type=jnp.float32)
         + jnp.dot(e_sol, wib[...], preferred_element_type=jnp.float32)
         + ci[...])
    inter = jnp.maximum(_ln(h, gi[...], bi[...]), 0.0)
    t = jnp.maximum(temp[...] * wt1[...] + ct1[...], 0.0)
    temp_emb = jnp.dot(t, wt2[...], preferred_element_type=jnp.float32) + ct2[...]
    f = (jnp.dot(e_aq, wf1a[...], preferred_element_type=jnp.float32)
         + jnp.dot(inter, wf1b[...], preferred_element_type=jnp.float32)
         + jnp.dot(temp_emb, wf1c[...], preferred_element_type=jnp.float32)
         + cf1[...])
    f = jnp.maximum(_ln(f, gf1[...], bf1[...]), 0.0)
    g2 = jnp.dot(f, wf2[...], preferred_element_type=jnp.float32) + cf2[...]
    g2 = jnp.maximum(_ln(g2, gf2[...], bf2[...]), 0.0)
    o_ref[...] = (jnp.sum(g2 * wf3t[...], axis=-1, keepdims=True) + cf3[...])


def _head(args):
    return pl.pallas_call(
        _head_body,
        out_shape=jax.ShapeDtypeStruct((NGRAPHS, 1), jnp.float32),
    )(*args)


_BN_INV = float(1.0 / (1.0 + 1e-5) ** 0.5)


def _fold_layer(lp):
    s1 = lp["g1"] * _BN_INV
    s2 = lp["g2"] * _BN_INV
    return (lp["eps"].astype(jnp.float32),
            lp["W1"] * s1[None, :], lp["b1"] * s1 + lp["c1"],
            lp["W2"] * s2[None, :], lp["b2"] * s2 + lp["c2"])


def _encoder(enc, x_pad, src, dst, batch_pad):
    x = _embed(x_pad, enc["We"], enc["be"])
    for lp in enc["layers"]:
        part = _agg(src, dst, x)
        eps, w1, b1, w2, b2 = _fold_layer(lp)
        x = _mlp(x, part, eps, w1, b1, w2, b2)
    return _pool(x, batch_pad, jnp.ones((16, HID), jnp.float32))


def kernel(drug_x, drug_edge_index, drug_batch,
           solvent_x, solvent_edge_index, solvent_batch,
           temperature, params):
    f32 = jnp.float32
    i32 = jnp.int32

    def pad_rows(x):
        return jnp.concatenate(
            [x.astype(f32), jnp.zeros((NPADR - x.shape[0], x.shape[1]), f32)])

    def pad_edges(ei):
        src = jnp.concatenate(
            [ei[0].astype(i32), jnp.zeros((EPAD - ei.shape[1],), i32)])
        dst = jnp.concatenate(
            [ei[1].astype(i32), jnp.full((EPAD - ei.shape[1],), INVALID_DST)])
        return src, dst

    def pad_batch(b):
        return jnp.concatenate(
            [b.astype(i32), jnp.full((NPADR - b.shape[0],), i32(NGRAPHS))])

    dx = pad_rows(drug_x)
    sx = pad_rows(solvent_x)
    dsrc, ddst = pad_edges(drug_edge_index)
    ssrc, sdst = pad_edges(solvent_edge_index)
    dbatch = pad_batch(drug_batch)
    sbatch = pad_batch(solvent_batch)

    sa, ca = _encoder(params["enc_aq"], dx, dsrc, ddst, dbatch)
    sb, cb = _encoder(params["enc_bs"], dx, dsrc, ddst, dbatch)
    ss_, cs = _encoder(params["enc_sol"], sx, ssrc, sdst, sbatch)

    p = params
    wi = p["Wi"]
    wf1 = p["Wf1"]
    head_args = (
        sa, ca, sb, cb, ss_, cs, temperature.astype(f32),
        wi[:HID], wi[HID:], p["ci"].reshape(1, HID),
        p["gi"].reshape(1, HID), p["bi"].reshape(1, HID),
        p["Wt1"].reshape(1, 32), p["ct1"].reshape(1, 32),
        p["Wt2"], p["ct2"].reshape(1, 32),
        wf1[:HID], wf1[HID:2 * HID], wf1[2 * HID:],
        p["cf1"].reshape(1, HID),
        p["gf1"].reshape(1, HID), p["bf1"].reshape(1, HID),
        p["Wf2"], p["cf2"].reshape(1, HID // 2),
        p["gf2"].reshape(1, HID // 2), p["bf2"].reshape(1, HID // 2),
        p["Wf3"].reshape(1, HID // 2), p["cf3"].reshape(1, 1),
    )
    return _head(head_args)


# speedup vs baseline: 3.2581x; 3.1901x over previous
import functools

import jax
import jax.numpy as jnp
from jax import lax
from jax.experimental import pallas as pl
from jax.experimental.pallas import tpu as pltpu
from jax.experimental.pallas import tpu_sc as plsc

HID = 128
N_NODES = 50000
N_EDGES = 800000
NGRAPHS = 1024

NC = 2
NS = 16
NW = NC * NS

NPADR = 50176
RBLK = 1568
NCHUNK = 4
CH = 12544
CSL = CH // NS
EPAD = 819200
EQ = EPAD // NW
EB = 2560
NBUF = 2
PNB = 7
INVALID_DST = 1 << 30

@functools.lru_cache(maxsize=None)
def _sc_mesh():
    return plsc.VectorSubcoreMesh(
        core_axis_name="c", subcore_axis_name="s",
        num_cores=NC, num_subcores=NS)


def _agg(src, dst, x):
    return _make_agg()(src, dst, x)


GR = 64
NTB = EB // GR
NDUM = 16


@functools.lru_cache(maxsize=None)
def _make_agg():
    return functools.partial(
        pl.kernel,
        out_type=jax.ShapeDtypeStruct((NC, NPADR, HID), jnp.float32),
        mesh=_sc_mesh(),
        scratch_types=[
            pltpu.VMEM((EB,), jnp.int32),
            pltpu.VMEM((EB,), jnp.int32),
            pltpu.VMEM((NTB, GR), jnp.int32),
            pltpu.VMEM((NBUF, GR, HID), jnp.float32),
            pltpu.VMEM((16, HID), jnp.float32),
            pltpu.VMEM_SHARED((CH + NDUM, HID), jnp.float32),
            pltpu.SemaphoreType.DMA((NBUF,)),
            pltpu.SemaphoreType.DMA((NBUF,)),
        ],
    )(_agg_body)


def _agg_body(src_hbm, dst_hbm, x_hbm, out_hbm,
              ssbuf, sdbuf, idx2, rows, zbuf, acc, gsem, ssem):
    c = lax.axis_index("c")
    s = lax.axis_index("s")
    wid = s * NC + c
    ebase = wid * EQ
    lane = lax.iota(jnp.int32, 16)

    zv = jnp.zeros((16,), jnp.float32)

    def _zrow(r, carry):
        for k in range(HID // 16):
            zbuf[r, pl.ds(k * 16, 16)] = zv
        return carry
    lax.fori_loop(0, 16, _zrow, 0)

    def _chunk(ci, carry):
        base = ci * CH
        roff = s * CSL

        def _zacc(r, carry2):
            pltpu.sync_copy(zbuf, acc.at[pl.ds(roff + r * 16, 16), :])
            return carry2
        lax.fori_loop(0, CSL // 16, _zacc, 0)
        plsc.subcore_barrier()

        def _blk(bl, carry2):
            pltpu.sync_copy(src_hbm.at[pl.ds(ebase + bl * EB, EB)], ssbuf)
            pltpu.sync_copy(dst_hbm.at[pl.ds(ebase + bl * EB, EB)], sdbuf)

            def _repack(v, carry3):
                sdv = sdbuf[pl.ds(v * 16, 16)]
                m = (sdv >= base) & (sdv < base + CH)
                idx2[v // (GR // 16), pl.ds((v % (GR // 16)) * 16, 16)] = (
                    jnp.where(m, sdv - base, CH + lane))
                return carry3
            lax.fori_loop(0, EB // 16, _repack, 0)

            def _pair(t, carry3):
                gds = []
                for b in range(NBUF):
                    j = t * NBUF + b
                    gds.append(pltpu.async_copy(
                        x_hbm.at[ssbuf.at[pl.ds(j * GR, GR)]],
                        rows.at[b], gsem.at[b]))
                sds = []
                for b in range(NBUF):
                    gds[b].wait()
                    j = t * NBUF + b
                    sds.append(pltpu.async_copy(
                        rows.at[b], acc.at[idx2.at[j]], ssem.at[b], add=True))
                for d in sds:
                    d.wait()
                return carry3
            lax.fori_loop(0, NTB // NBUF, _pair, 0)
            return carry2
        lax.fori_loop(0, EQ // EB, _blk, 0)
        plsc.subcore_barrier()
        pltpu.sync_copy(acc.at[pl.ds(roff, CSL), :],
                        out_hbm.at[c, pl.ds(base + roff, CSL), :])
        plsc.subcore_barrier()
        return carry
    lax.fori_loop(0, NCHUNK, _chunk, 0)


def _pool(x, batch, ones):
    return _make_pool()(x, batch, ones)


@functools.lru_cache(maxsize=None)
def _make_pool():
    return functools.partial(
        pl.kernel,
        out_type=(jax.ShapeDtypeStruct((NC, NGRAPHS, HID), jnp.float32),
                  jax.ShapeDtypeStruct((NC, NGRAPHS, HID), jnp.float32)),
        mesh=_sc_mesh(),
        scratch_types=[
            pltpu.VMEM((RBLK,), jnp.int32),
            pltpu.VMEM((PNB, 16, HID), jnp.float32),
            pltpu.VMEM((16, HID), jnp.float32),
            pltpu.VMEM((64, HID), jnp.float32),
            pltpu.VMEM_SHARED((NGRAPHS + 1, HID), jnp.float32),
            pltpu.VMEM_SHARED((NGRAPHS + 1, HID), jnp.float32),
            pltpu.SemaphoreType.DMA((PNB,)),
            pltpu.SemaphoreType.DMA((PNB,)),
            pltpu.SemaphoreType.DMA((PNB,)),
        ],
    )(_pool_body)


def _pool_body(x_hbm, batch_hbm, ones_hbm, sums_hbm, cnts_hbm,
               bbuf, rows, ones, zs, accs, accc, gsem, ssem, csem):
    c = lax.axis_index("c")
    s = lax.axis_index("s")
    wid = s * NC + c
    nbase = wid * RBLK
    pltpu.sync_copy(batch_hbm.at[pl.ds(nbase, RBLK)], bbuf)
    pltpu.sync_copy(ones_hbm, ones)

    zv = jnp.zeros((16,), jnp.float32)

    def _zrow(r, carry):
        for k in range(HID // 16):
            zs[r, pl.ds(k * 16, 16)] = zv
        return carry
    lax.fori_loop(0, 64, _zrow, 0)

    goff = s * (NGRAPHS // NS)
    pltpu.sync_copy(zs, accs.at[pl.ds(goff, 64), :])
    pltpu.sync_copy(zs, accc.at[pl.ds(goff, 64), :])
    plsc.subcore_barrier()

    def _ring(t, carry):
        gds = []
        for b in range(PNB):
            noff = nbase + (t * PNB + b) * 16
            gds.append(pltpu.async_copy(
                x_hbm.at[pl.ds(noff, 16), :], rows.at[b], gsem.at[b]))
        sds = []
        for b in range(PNB):
            gds[b].wait()
            idxv = bbuf[pl.ds((t * PNB + b) * 16, 16)]
            sds.append(pltpu.async_copy(
                rows.at[b], accs.at[idxv], ssem.at[b], add=True))
            sds.append(pltpu.async_copy(
                ones, accc.at[idxv], csem.at[b], add=True))
        for d in sds:
            d.wait()
        return carry
    lax.fori_loop(0, (RBLK // 16) // PNB, _ring, 0)
    plsc.subcore_barrier()
    pltpu.sync_copy(accs.at[pl.ds(goff, 64), :],
                    sums_hbm.at[c, pl.ds(goff, 64), :])
    pltpu.sync_copy(accc.at[pl.ds(goff, 64), :],
                    cnts_hbm.at[c, pl.ds(goff, 64), :])


def _embed_body(x_ref, w_ref, b_ref, o_ref):
    o_ref[...] = jnp.dot(x_ref[...], w_ref[...],
                         preferred_element_type=jnp.float32) + b_ref[...]


def _embed(x, w, b):
    nfeat = x.shape[1]
    return pl.pallas_call(
        _embed_body,
        grid=(NPADR // RBLK,),
        in_specs=[pl.BlockSpec((RBLK, nfeat), lambda i: (i, 0)),
                  pl.BlockSpec((nfeat, HID), lambda i: (0, 0)),
                  pl.BlockSpec((1, HID), lambda i: (0, 0))],
        out_specs=pl.BlockSpec((RBLK, HID), lambda i: (i, 0)),
        out_shape=jax.ShapeDtypeStruct((NPADR, HID), jnp.float32),
    )(x, w, b.reshape(1, HID))


def _mlp_body(x_ref, p_ref, eps_ref, w1_ref, b1_ref, w2_ref, b2_ref, o_ref):
    h = x_ref[...] * (eps_ref[...] + 1.0) + p_ref[0] + p_ref[1]
    a = jnp.dot(h, w1_ref[...], preferred_element_type=jnp.float32)
    a = jnp.maximum(a + b1_ref[...], 0.0)
    o = jnp.dot(a, w2_ref[...], preferred_element_type=jnp.float32)
    o_ref[...] = jnp.maximum(o + b2_ref[...], 0.0)


def _mlp(x, part, eps, w1, b1, w2, b2):
    return pl.pallas_call(
        _mlp_body,
        grid=(NPADR // RBLK,),
        in_specs=[pl.BlockSpec((RBLK, HID), lambda i: (i, 0)),
                  pl.BlockSpec((NC, RBLK, HID), lambda i: (0, i, 0)),
                  pl.BlockSpec((1, 1), lambda i: (0, 0)),
                  pl.BlockSpec((HID, 2 * HID), lambda i: (0, 0)),
                  pl.BlockSpec((1, 2 * HID), lambda i: (0, 0)),
                  pl.BlockSpec((2 * HID, HID), lambda i: (0, 0)),
                  pl.BlockSpec((1, HID), lambda i: (0, 0))],
        out_specs=pl.BlockSpec((RBLK, HID), lambda i: (i, 0)),
        out_shape=jax.ShapeDtypeStruct((NPADR, HID), jnp.float32),
    )(x, part, eps.reshape(1, 1), w1, b1.reshape(1, 2 * HID),
      w2, b2.reshape(1, HID))


def _ln(x, g, b):
    m = jnp.mean(x, axis=-1, keepdims=True)
    xm = x - m
    v = jnp.mean(xm * xm, axis=-1, keepdims=True)
    return xm / jnp.sqrt(v + 1e-5) * g + b


def _head_body(sa, ca, sb, cb, ss_, cs, temp, wia, wib, ci, gi, bi,
               wt1, ct1, wt2, ct2, wf1a, wf1b, wf1c, cf1, gf1, bf1,
               wf2, cf2, gf2, bf2, wf3t, cf3, o_ref):
    def emb(s_ref, c_ref):
        ssum = s_ref[0] + s_ref[1]
        cnt = c_ref[0][:, 0:1] + c_ref[1][:, 0:1]
        return ssum / jnp.maximum(cnt, 1.0)

    e_aq = emb(sa, ca)
    e_bs = emb(sb, cb)
    e_sol = emb(ss_, cs)
    h = (jnp.dot(e_bs, wia[...], preferred_element_type=jnp.float32)
         + jnp.dot(e_sol, wib[...], preferred_element_type=jnp.float32)
         + ci[...])
    inter = jnp.maximum(_ln(h, gi[...], bi[...]), 0.0)
    t = jnp.maximum(temp[...] * wt1[...] + ct1[...], 0.0)
    temp_emb = jnp.dot(t, wt2[...], preferred_element_type=jnp.float32) + ct2[...]
    f = (jnp.dot(e_aq, wf1a[...], preferred_element_type=jnp.float32)
         + jnp.dot(inter, wf1b[...], preferred_element_type=jnp.float32)
         + jnp.dot(temp_emb, wf1c[...], preferred_element_type=jnp.float32)
         + cf1[...])
    f = jnp.maximum(_ln(f, gf1[...], bf1[...]), 0.0)
    g2 = jnp.dot(f, wf2[...], preferred_element_type=jnp.float32) + cf2[...]
    g2 = jnp.maximum(_ln(g2, gf2[...], bf2[...]), 0.0)
    o_ref[...] = (jnp.sum(g2 * wf3t[...], axis=-1, keepdims=True) + cf3[...])


def _head(args):
    return pl.pallas_call(
        _head_body,
        out_shape=jax.ShapeDtypeStruct((NGRAPHS, 1), jnp.float32),
    )(*args)


_BN_INV = float(1.0 / (1.0 + 1e-5) ** 0.5)


def _fold_layer(lp):
    s1 = lp["g1"] * _BN_INV
    s2 = lp["g2"] * _BN_INV
    return (lp["eps"].astype(jnp.float32),
            lp["W1"] * s1[None, :], lp["b1"] * s1 + lp["c1"],
            lp["W2"] * s2[None, :], lp["b2"] * s2 + lp["c2"])


def _encoder(enc, x_pad, src, dst, batch_pad):
    x = _embed(x_pad, enc["We"], enc["be"])
    for lp in enc["layers"]:
        part = _agg(src, dst, x)
        eps, w1, b1, w2, b2 = _fold_layer(lp)
        x = _mlp(x, part, eps, w1, b1, w2, b2)
    return _pool(x, batch_pad, jnp.ones((16, HID), jnp.float32))


def kernel(drug_x, drug_edge_index, drug_batch,
           solvent_x, solvent_edge_index, solvent_batch,
           temperature, params):
    f32 = jnp.float32
    i32 = jnp.int32

    def pad_rows(x):
        return jnp.concatenate(
            [x.astype(f32), jnp.zeros((NPADR - x.shape[0], x.shape[1]), f32)])

    def pad_edges(ei):
        e = ei.shape[1]
        eq0 = e // NW
        padw = (EPAD - e) // NW
        src = ei[0].astype(i32).reshape(NW, eq0)
        dst = ei[1].astype(i32).reshape(NW, eq0)
        psrc = jnp.broadcast_to(
            ((jnp.arange(padw, dtype=i32) * 83) % N_NODES)[None], (NW, padw))
        pdst = jnp.full((NW, padw), INVALID_DST, i32)
        src = jnp.concatenate([src, psrc], axis=1).reshape(-1)
        dst = jnp.concatenate([dst, pdst], axis=1).reshape(-1)
        return src, dst

    def pad_batch(b):
        return jnp.concatenate(
            [b.astype(i32), jnp.full((NPADR - b.shape[0],), i32(NGRAPHS))])

    dx = pad_rows(drug_x)
    sx = pad_rows(solvent_x)
    dsrc, ddst = pad_edges(drug_edge_index)
    ssrc, sdst = pad_edges(solvent_edge_index)
    dbatch = pad_batch(drug_batch)
    sbatch = pad_batch(solvent_batch)

    sa, ca = _encoder(params["enc_aq"], dx, dsrc, ddst, dbatch)
    sb, cb = _encoder(params["enc_bs"], dx, dsrc, ddst, dbatch)
    ss_, cs = _encoder(params["enc_sol"], sx, ssrc, sdst, sbatch)

    p = params
    wi = p["Wi"]
    wf1 = p["Wf1"]
    head_args = (
        sa, ca, sb, cb, ss_, cs, temperature.astype(f32),
        wi[:HID], wi[HID:], p["ci"].reshape(1, HID),
        p["gi"].reshape(1, HID), p["bi"].reshape(1, HID),
        p["Wt1"].reshape(1, 32), p["ct1"].reshape(1, 32),
        p["Wt2"], p["ct2"].reshape(1, 32),
        wf1[:HID], wf1[HID:2 * HID], wf1[2 * HID:],
        p["cf1"].reshape(1, HID),
        p["gf1"].reshape(1, HID), p["bf1"].reshape(1, HID),
        p["Wf2"], p["cf2"].reshape(1, HID // 2),
        p["gf2"].reshape(1, HID // 2), p["bf2"].reshape(1, HID // 2),
        p["Wf3"].reshape(1, HID // 2), p["cf3"].reshape(1, 1),
    )
    return _head(head_args)
